# Initial kernel scaffold; baseline (speedup 1.0000x reference)
import jax
import jax.numpy as jnp
from jax.experimental import pallas as pl


def kernel(x, edge_index, batch, edge_weight, params):
    raise NotImplementedError("write your pallas kernel here")


# speedup vs baseline: 54.4713x; 54.4713x over previous
import functools

import jax
import jax.numpy as jnp
from jax import lax
from jax.experimental import pallas as pl
from jax.experimental.pallas import tpu as pltpu
from jax.experimental.pallas import tpu_sc as plsc

N = 100000
E = 6400000
HH = 16
NG = 512

NPAD = 100352
ROWS_E = E // 128
CJ = 16
CHUNKS_E = ROWS_E // CJ
NW = 32
LOOP_E = (CHUNKS_E + NW - 1) // NW
PSUB = NPAD // 16
NROWS_N = NPAD // 128
LOOP_N = (NROWS_N // 8 + NW - 1) // NW
NSEGP = 1024
NHALF = NPAD // 2
PSUBH = NHALF // 16
ZNH = PSUBH // 8
LOOP_E2 = (CHUNKS_E + 15) // 16

_mesh = plsc.VectorSubcoreMesh(core_axis_name="c", subcore_axis_name="s",
                               num_cores=2, num_subcores=16)
_cparams = pltpu.CompilerParams(use_tc_tiling_on_sc=False)
f32 = jnp.float32
i32 = jnp.int32


def _wid():
    return lax.axis_index("s") * 2 + lax.axis_index("c")


@functools.partial(
    pl.kernel,
    out_type=(
        jax.ShapeDtypeStruct((2 * NPAD,), f32),
        jax.ShapeDtypeStruct((ROWS_E, 128), f32),
    ),
    mesh=_mesh,
    compiler_params=_cparams,
    scratch_types=[
        pltpu.VMEM((CJ, 128), i32),
        pltpu.VMEM((CJ, 128), i32),
        pltpu.VMEM((CJ, 128), f32),
        pltpu.VMEM((CJ, 128), f32),
        pltpu.VMEM((CJ, 128), f32),
        pltpu.VMEM((CJ, 128), f32),
        pltpu.VMEM((CJ, 128), f32),
        pltpu.VMEM((PSUB,), f32),
        pltpu.VMEM_SHARED((NPAD,), f32),
        pltpu.SemaphoreType.DMA,
        pltpu.SemaphoreType.DMA,
    ],
)
def _edge_pass_a(src2, dst2, ew2, es_h, ed_h, dp, eew2,
                 sv, dv, wv, eg, dg, ee, ewo, zb, den_sh, semA, semB):
    c = lax.axis_index("c")
    s = lax.axis_index("s")
    wid = _wid()

    def zinit(i, _):
        zb[pl.ds(i * 16, 16)] = jnp.zeros((16,), f32)
        return 0
    lax.fori_loop(0, PSUB // 16, zinit, 0)
    pltpu.sync_copy(zb, den_sh.at[pl.ds(s * PSUB, PSUB)])
    plsc.subcore_barrier()

    def chunk(i, _):
        cid = i * NW + wid

        @pl.when(cid < CHUNKS_E)
        def _():
            row = cid * CJ
            pltpu.sync_copy(src2.at[pl.ds(row, CJ)], sv)
            pltpu.sync_copy(dst2.at[pl.ds(row, CJ)], dv)
            pltpu.sync_copy(ew2.at[pl.ds(row, CJ)], wv)
            hs = [pltpu.async_copy(es_h.at[sv.at[j]], eg.at[j], semA)
                  for j in range(CJ)]
            hd = [pltpu.async_copy(ed_h.at[dv.at[j]], dg.at[j], semB)
                  for j in range(CJ)]
            for h in hs:
                h.wait()
            for h in hd:
                h.wait()

            def comp(k, _):
                j = k // 8
                l = (k % 8) * 16
                a = eg[j, pl.ds(l, 16)] + dg[j, pl.ds(l, 16)]
                e = jnp.where(a > 0.0, a, 0.2 * a)
                x = jnp.exp(e)
                ee[j, pl.ds(l, 16)] = x
                ewo[j, pl.ds(l, 16)] = x * wv[j, pl.ds(l, 16)]
                return 0
            lax.fori_loop(0, CJ * 8, comp, 0)

            for j in range(CJ):
                pltpu.sync_copy(ee.at[j], den_sh.at[dv.at[j]], add=True)
            pltpu.sync_copy(ewo, eew2.at[pl.ds(row, CJ)])
        return 0
    lax.fori_loop(0, LOOP_E, chunk, 0)
    plsc.subcore_barrier()
    pltpu.sync_copy(den_sh.at[pl.ds(s * PSUB, PSUB)], zb)
    pltpu.sync_copy(zb, dp.at[pl.ds(c * NPAD + s * PSUB, PSUB)])


@functools.partial(
    pl.kernel,
    out_type=jax.ShapeDtypeStruct((NPAD, HH), f32),
    mesh=_mesh,
    compiler_params=_cparams,
    scratch_types=[
        pltpu.VMEM((CJ, 128), i32),
        pltpu.VMEM((CJ, 128), i32),
        pltpu.VMEM((CJ, 128), i32),
        pltpu.VMEM((CJ, 128), f32),
        pltpu.VMEM((CJ, 128), f32),
        pltpu.VMEM((CJ * 128 + 16,), f32),
        pltpu.VMEM((CJ, 128, HH), f32),
        pltpu.VMEM((ZNH, HH), f32),
        pltpu.VMEM_SHARED((NHALF + 16, HH), f32),
        pltpu.SemaphoreType.DMA,
        pltpu.SemaphoreType.DMA,
    ],
)
def _edge_pass_b(src2, dst2, eew2, den_h, h_h, op,
                 sv, dv, lv, wv, dg, al, rows, zr, out_sh, semA, semB):
    c = lax.axis_index("c")
    s = lax.axis_index("s")
    base_node = c * NHALF

    def zinit(i, _):
        zr[i, :] = jnp.zeros((HH,), f32)
        return 0
    lax.fori_loop(0, ZNH, zinit, 0)
    for t in range(8):
        pltpu.sync_copy(zr, out_sh.at[pl.ds(s * PSUBH + t * ZNH, ZNH)])
    plsc.subcore_barrier()

    def chunk(i, _):
        cid = i * 16 + s

        @pl.when(cid < CHUNKS_E)
        def _():
            row = cid * CJ
            pltpu.sync_copy(src2.at[pl.ds(row, CJ)], sv)
            pltpu.sync_copy(dst2.at[pl.ds(row, CJ)], dv)
            pltpu.sync_copy(eew2.at[pl.ds(row, CJ)], wv)
            hd = [pltpu.async_copy(den_h.at[dv.at[j]], dg.at[j], semA)
                  for j in range(CJ)]
            hr = [pltpu.async_copy(h_h.at[sv.at[j]], rows.at[j], semB)
                  for j in range(CJ)]
            for h in hd:
                h.wait()
            for h in hr:
                h.wait()

            def compa(k, _):
                j = k // 8
                l = (k % 8) * 16
                dd = jnp.maximum(dg[j, pl.ds(l, 16)], 1e-16)
                al[pl.ds(j * 128 + l, 16)] = wv[j, pl.ds(l, 16)] / dd
                li = dv[j, pl.ds(l, 16)] - base_node
                oob = (li < 0) | (li >= NHALF)
                lv[j, pl.ds(l, 16)] = jnp.where(oob, NHALF, li)
                return 0
            lax.fori_loop(0, CJ * 8, compa, 0)

            def scale_j(j, _):
                base = j * 128

                def scale_r(r, _):
                    av = al[pl.ds(base + r, 16)]
                    rows[j, r, :] = rows[j, r, :] * av[0]
                    return 0
                lax.fori_loop(0, 128, scale_r, 0)
                return 0
            lax.fori_loop(0, CJ, scale_j, 0)

            for j in range(CJ):
                pltpu.sync_copy(rows.at[j], out_sh.at[lv.at[j]], add=True)
        return 0
    lax.fori_loop(0, LOOP_E2, chunk, 0)
    plsc.subcore_barrier()
    for t in range(8):
        pltpu.sync_copy(out_sh.at[pl.ds(s * PSUBH + t * ZNH, ZNH)], zr)
        pltpu.sync_copy(zr, op.at[pl.ds(base_node + s * PSUBH + t * ZNH, ZNH)])


@functools.partial(
    pl.kernel,
    out_type=(
        jax.ShapeDtypeStruct((2 * NSEGP, HH), f32),
        jax.ShapeDtypeStruct((2 * NSEGP,), f32),
    ),
    mesh=_mesh,
    compiler_params=_cparams,
    scratch_types=[
        pltpu.VMEM((8, 128), i32),
        pltpu.VMEM((1024, HH), f32),
        pltpu.VMEM((128,), f32),
        pltpu.VMEM((NSEGP // 16, HH), f32),
        pltpu.VMEM((NSEGP // 16,), f32),
        pltpu.VMEM_SHARED((NSEGP, HH), f32),
        pltpu.VMEM_SHARED((NSEGP,), f32),
        pltpu.SemaphoreType.DMA,
    ],
)
def _pool(xr, b2, pp, cp, bidx, rows, ones, zr, zc, pool_sh, cnt_sh, sem):
    c = lax.axis_index("c")
    s = lax.axis_index("s")
    wid = _wid()
    zn = NSEGP // 16

    def zinit(i, _):
        zr[i, :] = jnp.zeros((HH,), f32)
        return 0
    lax.fori_loop(0, zn, zinit, 0)

    def zinit2(i, _):
        zc[pl.ds(i * 16, 16)] = jnp.zeros((16,), f32)
        return 0
    lax.fori_loop(0, zn // 16, zinit2, 0)

    def oinit(i, _):
        ones[pl.ds(i * 16, 16)] = jnp.ones((16,), f32)
        return 0
    lax.fori_loop(0, 8, oinit, 0)

    pltpu.sync_copy(zr, pool_sh.at[pl.ds(s * zn, zn)])
    pltpu.sync_copy(zc, cnt_sh.at[pl.ds(s * zn, zn)])
    plsc.subcore_barrier()

    def body(i, _):
        g = i * NW + wid

        @pl.when(g < NROWS_N // 8)
        def _():
            r = g * 8
            pltpu.sync_copy(b2.at[pl.ds(r, 8)], bidx)
            pltpu.sync_copy(xr.at[pl.ds(r * 128, 1024)], rows)
            for t in range(8):
                pltpu.sync_copy(rows.at[pl.ds(t * 128, 128)],
                                pool_sh.at[bidx.at[t]], add=True)
                pltpu.sync_copy(ones, cnt_sh.at[bidx.at[t]], add=True)
        return 0
    lax.fori_loop(0, LOOP_N, body, 0)
    plsc.subcore_barrier()
    pltpu.sync_copy(pool_sh.at[pl.ds(s * zn, zn)], zr)
    pltpu.sync_copy(zr, pp.at[pl.ds(c * NSEGP + s * zn, zn)])
    pltpu.sync_copy(cnt_sh.at[pl.ds(s * zn, zn)], zc)
    pltpu.sync_copy(zc, cp.at[pl.ds(c * NSEGP + s * zn, zn)])


def _graphnorm(x, w, b, ms, eps=1e-5):
    mean = jnp.mean(x, axis=0, keepdims=True)
    out = x - ms * mean
    var = jnp.mean(out * out, axis=0, keepdims=True)
    return out / jnp.sqrt(var + eps) * w + b


def _gat_layer(act, src2, dst2, ew2, W, a_s, a_d, b):
    h = act @ W
    es = h @ a_s
    ed = h @ a_d
    hp = jnp.zeros((NPAD, HH), f32).at[:N].set(h)
    esp = jnp.zeros((NPAD,), f32).at[:N].set(es)
    edp = jnp.zeros((NPAD,), f32).at[:N].set(ed)
    dp, eew2 = _edge_pass_a(src2, dst2, ew2, esp, edp)
    den = dp[:NPAD] + dp[NPAD:]
    op = _edge_pass_b(src2, dst2, eew2, den, hp)
    return op[:N] + b


@jax.jit
def kernel(x, edge_index, batch, edge_weight, params):
    p = params
    src2 = edge_index[0].reshape(ROWS_E, 128)
    dst2 = edge_index[1].reshape(ROWS_E, 128)
    ew2 = edge_weight.reshape(ROWS_E, 128)
    b2 = jnp.full((NPAD,), NG, i32).at[:N].set(batch).reshape(NROWS_N, 128)

    agg = _gat_layer(x, src2, dst2, ew2, p["W1"], p["as1"], p["ad1"], p["b1"])
    x1 = jax.nn.relu(_graphnorm(agg, p["gnw1"], p["gnb1"], p["gnm1"]))
    xi = x1
    for l in (2, 3, 4):
        agg = _gat_layer(xi, src2, dst2, ew2,
                         p[f"W{l}"], p[f"as{l}"], p[f"ad{l}"], p[f"b{l}"])
        h = _graphnorm(agg, p[f"gnw{l}"], p[f"gnb{l}"], p[f"gnm{l}"])
        xi = jax.nn.relu(h + xi)

    xr = jnp.zeros((NPAD, HH), f32).at[:N].set(xi)
    pp, cp = _pool(xr, b2)
    pooled = (pp[:NSEGP] + pp[NSEGP:])[:NG]
    counts = (cp[:NSEGP] + cp[NSEGP:])[:NG]
    pooled = pooled / jnp.maximum(counts, 1.0)[:, None]
    return pooled @ p["Wl"] + p["bl"]
